# Initial kernel scaffold; baseline (speedup 1.0000x reference)
import jax
import jax.numpy as jnp
from jax.experimental import pallas as pl


def kernel(input_wordids, near_wordids, neg_wordids, input_weight):
    raise NotImplementedError("write your pallas kernel here")


# speedup vs baseline: 5.9044x; 5.9044x over previous
import functools

import jax
import jax.numpy as jnp
from jax import lax
from jax.experimental import pallas as pl
from jax.experimental.pallas import tpu as pltpu
from jax.experimental.pallas import tpu_sc as plsc

NC = 2
NS = 16
NW = NC * NS
LANES = 16


def _sc_logits(B, P, N, V, D, W, GCH):
    BPT = B // NW
    NCHUNK = BPT // W
    NBUF = 2
    WP, WN = W * P, W * N

    mesh = plsc.VectorSubcoreMesh(core_axis_name="c", subcore_axis_name="s")

    scratch = []
    for _ in range(NBUF):
        scratch += [
            pltpu.VMEM((W,), jnp.int32),
            pltpu.VMEM((WP,), jnp.int32),
            pltpu.VMEM((WN,), jnp.int32),
            pltpu.VMEM((W, D), jnp.float32),
            pltpu.VMEM((WP, D), jnp.float32),
            pltpu.VMEM((WN, D), jnp.float32),
            pltpu.SemaphoreType.DMA,
        ]
    scratch += [
        pltpu.VMEM((BPT * P,), jnp.float32),
        pltpu.VMEM((BPT * N,), jnp.float32),
    ]

    @functools.partial(
        pl.kernel,
        out_type=(
            jax.ShapeDtypeStruct((B * P,), jnp.float32),
            jax.ShapeDtypeStruct((B * N,), jnp.float32),
        ),
        mesh=mesh,
        scratch_types=scratch,
        compiler_params=pltpu.CompilerParams(needs_layout_passes=False,
                                             use_tc_tiling_on_sc=False),
    )
    def k(inp_ids, near_ids, neg_ids, table, out_near, out_neg, *s):
        bufs = [s[i * 7:(i + 1) * 7] for i in range(NBUF)]
        ln_all, lg_all = s[NBUF * 7], s[NBUF * 7 + 1]
        wid = lax.axis_index("s") * NC + lax.axis_index("c")
        base = wid * BPT

        def fire(c, r):
            idx_i, idx_p, idx_n, rows_i, rows_p, rows_n, sem = bufs[r]
            b0 = base + c * W
            pltpu.sync_copy(inp_ids.at[pl.ds(b0, W)], idx_i)
            pltpu.sync_copy(near_ids.at[pl.ds(b0 * P, WP)], idx_p)
            pltpu.sync_copy(neg_ids.at[pl.ds(b0 * N, WN)], idx_n)
            pltpu.async_copy(table.at[idx_i], rows_i, sem)
            for o in range(0, WP, GCH):
                pltpu.async_copy(table.at[idx_p.at[pl.ds(o, GCH)]],
                                 rows_p.at[pl.ds(o, GCH)], sem)
            for o in range(0, WN, GCH):
                pltpu.async_copy(table.at[idx_n.at[pl.ds(o, GCH)]],
                                 rows_n.at[pl.ds(o, GCH)], sem)

        def drain(r):
            idx_i, idx_p, idx_n, rows_i, rows_p, rows_n, sem = bufs[r]
            pltpu.make_async_copy(table.at[idx_i], rows_i, sem).wait()
            for o in range(0, WP, GCH):
                pltpu.make_async_copy(table.at[idx_p.at[pl.ds(o, GCH)]],
                                      rows_p.at[pl.ds(o, GCH)], sem).wait()
            for o in range(0, WN, GCH):
                pltpu.make_async_copy(table.at[idx_n.at[pl.ds(o, GCH)]],
                                      rows_n.at[pl.ds(o, GCH)], sem).wait()

        lane = lax.iota(jnp.int32, LANES)
        last = lane == (LANES - 1)

        def compute(c, r):
            _, _, _, rows_i, rows_p, rows_n, _ = bufs[r]

            def per_b(b, _):
                ivecs = [rows_i[b, pl.ds(j * LANES, LANES)]
                         for j in range(D // LANES)]

                def dots(K, rows, out):
                    def per_k(kk, _):
                        row = b * K + kk
                        acc = rows[row, pl.ds(0, LANES)] * ivecs[0]
                        for j in range(1, D // LANES):
                            acc = acc + rows[row, pl.ds(j * LANES, LANES)] * ivecs[j]
                        s = plsc.cumsum(acc)
                        oidx = jnp.full((LANES,), (c * W + b) * K + kk,
                                        jnp.int32)
                        plsc.store_scatter(out, [oidx], s, mask=last)
                        return 0
                    lax.fori_loop(0, K, per_k, 0)

                dots(P, rows_p, ln_all)
                dots(N, rows_n, lg_all)
                return 0

            lax.fori_loop(0, W, per_b, 0)

        fire(0, 0)

        def step(i, _):
            for r in range(NBUF):
                c = i * NBUF + r
                cn = jnp.minimum(c + 1, NCHUNK - 1)
                fire(cn, (r + 1) % NBUF)
                drain(r)
                compute(c, r)
            return 0

        lax.fori_loop(0, NCHUNK // NBUF, step, 0)
        drain(0)

        pltpu.sync_copy(ln_all, out_near.at[pl.ds(base * P, BPT * P)])
        pltpu.sync_copy(lg_all, out_neg.at[pl.ds(base * N, BPT * N)])

    return k


def _tc_loss(ln, lg):
    B, P = ln.shape
    N = lg.shape[1]
    BLK = 2048

    def body(ln_ref, lg_ref, out_ref):
        def lsig(x):
            return jnp.minimum(x, 0.0) - jnp.log1p(jnp.exp(-jnp.abs(x)))
        out_ref[...] = -(lsig(ln_ref[...]).sum(axis=1)
                         + lsig(-lg_ref[...]).sum(axis=1))

    return pl.pallas_call(
        body,
        grid=(B // BLK,),
        in_specs=[
            pl.BlockSpec((BLK, P), lambda i: (i, 0)),
            pl.BlockSpec((BLK, N), lambda i: (i, 0)),
        ],
        out_specs=pl.BlockSpec((BLK,), lambda i: (i,)),
        out_shape=jax.ShapeDtypeStruct((B,), jnp.float32),
    )(ln, lg)


def kernel(input_wordids, near_wordids, neg_wordids, input_weight):
    B, P = near_wordids.shape
    N = neg_wordids.shape[1]
    V, D = input_weight.shape
    W = 8
    GCH = 80

    ids = input_wordids.astype(jnp.int32)
    near = near_wordids.reshape(B * P).astype(jnp.int32)
    neg = neg_wordids.reshape(B * N).astype(jnp.int32)

    ln, lg = _sc_logits(B, P, N, V, D, W, GCH)(ids, near, neg, input_weight)
    return _tc_loss(ln.reshape(B, P), lg.reshape(B, N))


# speedup vs baseline: 9.1380x; 1.5477x over previous
import functools

import jax
import jax.numpy as jnp
from jax import lax
from jax.experimental import pallas as pl
from jax.experimental.pallas import tpu as pltpu
from jax.experimental.pallas import tpu_sc as plsc

NC = 2
NS = 16
NW = NC * NS
LANES = 16


def _sc_logits(B, P, N, V, D, W, GCH, UNROLL):
    BPT = B // NW
    NCHUNK = BPT // W
    NBUF = 2
    WP, WN = W * P, W * N

    mesh = plsc.VectorSubcoreMesh(core_axis_name="c", subcore_axis_name="s")

    scratch = []
    for _ in range(NBUF):
        scratch += [
            pltpu.VMEM((W,), jnp.int32),
            pltpu.VMEM((WP,), jnp.int32),
            pltpu.VMEM((WN,), jnp.int32),
            pltpu.VMEM((W, D), jnp.float32),
            pltpu.VMEM((WP, D), jnp.float32),
            pltpu.VMEM((WN, D), jnp.float32),
            pltpu.SemaphoreType.DMA,
            pltpu.SemaphoreType.DMA,
        ]
    scratch += [
        pltpu.VMEM((BPT * P,), jnp.float32),
        pltpu.VMEM((BPT * N,), jnp.float32),
    ]

    @functools.partial(
        pl.kernel,
        out_type=(
            jax.ShapeDtypeStruct((B * P,), jnp.float32),
            jax.ShapeDtypeStruct((B * N,), jnp.float32),
        ),
        mesh=mesh,
        scratch_types=scratch,
        compiler_params=pltpu.CompilerParams(needs_layout_passes=False,
                                             use_tc_tiling_on_sc=False),
    )
    def k(inp_ids, near_ids, neg_ids, table, out_near, out_neg, *s):
        bufs = [s[i * 8:(i + 1) * 8] for i in range(NBUF)]
        ln_all, lg_all = s[NBUF * 8], s[NBUF * 8 + 1]
        wid = lax.axis_index("s") * NC + lax.axis_index("c")
        base = wid * BPT
        LAST = NCHUNK - 1

        def fire_idx(c, r):
            idx_i, idx_p, idx_n = bufs[r][0:3]
            isem = bufs[r][7]
            b0 = base + c * W
            pltpu.async_copy(inp_ids.at[pl.ds(b0, W)], idx_i, isem)
            pltpu.async_copy(near_ids.at[pl.ds(b0 * P, WP)], idx_p, isem)
            pltpu.async_copy(neg_ids.at[pl.ds(b0 * N, WN)], idx_n, isem)

        def wait_idx(r):
            idx_i, idx_p, idx_n = bufs[r][0:3]
            isem = bufs[r][7]
            pltpu.make_async_copy(inp_ids.at[pl.ds(0, W)], idx_i, isem).wait()
            pltpu.make_async_copy(near_ids.at[pl.ds(0, WP)], idx_p, isem).wait()
            pltpu.make_async_copy(neg_ids.at[pl.ds(0, WN)], idx_n, isem).wait()

        def fire_gathers(r):
            idx_i, idx_p, idx_n, rows_i, rows_p, rows_n, gsem, _ = bufs[r]
            pltpu.async_copy(table.at[idx_i], rows_i, gsem)
            for o in range(0, WP, GCH):
                pltpu.async_copy(table.at[idx_p.at[pl.ds(o, GCH)]],
                                 rows_p.at[pl.ds(o, GCH)], gsem)
            for o in range(0, WN, GCH):
                pltpu.async_copy(table.at[idx_n.at[pl.ds(o, GCH)]],
                                 rows_n.at[pl.ds(o, GCH)], gsem)

        def drain_gathers(r):
            idx_i, idx_p, idx_n, rows_i, rows_p, rows_n, gsem, _ = bufs[r]
            pltpu.make_async_copy(table.at[idx_i], rows_i, gsem).wait()
            for o in range(0, WP, GCH):
                pltpu.make_async_copy(table.at[idx_p.at[pl.ds(o, GCH)]],
                                      rows_p.at[pl.ds(o, GCH)], gsem).wait()
            for o in range(0, WN, GCH):
                pltpu.make_async_copy(table.at[idx_n.at[pl.ds(o, GCH)]],
                                      rows_n.at[pl.ds(o, GCH)], gsem).wait()

        lane = lax.iota(jnp.int32, LANES)
        last_lane = lane == (LANES - 1)

        def compute(c, r):
            rows_i, rows_p, rows_n = bufs[r][3:6]
            for b in range(W):
                ivecs = [rows_i[b, pl.ds(j * LANES, LANES)]
                         for j in range(D // LANES)]

                def dots(K, rows, out, obase):
                    @plsc.parallel_loop(0, K, 1, unroll=UNROLL)
                    def _(kk):
                        row = b * K + kk
                        acc = rows[row, pl.ds(0, LANES)] * ivecs[0]
                        for j in range(1, D // LANES):
                            acc = acc + (rows[row, pl.ds(j * LANES, LANES)]
                                         * ivecs[j])
                        s_ = plsc.cumsum(acc)
                        oidx = jnp.full((LANES,), obase + kk, jnp.int32)
                        plsc.store_scatter(out, [oidx], s_, mask=last_lane)

                dots(P, rows_p, ln_all, (c * W + b) * P)
                dots(N, rows_n, lg_all, (c * W + b) * N)

        fire_idx(0, 0)
        wait_idx(0)
        fire_gathers(0)
        fire_idx(1, 1)

        def step(i, _):
            for r in range(NBUF):
                c = i * NBUF + r
                drain_gathers(r)
                fire_idx(jnp.minimum(c + 2, LAST), r)
                wait_idx(r ^ 1)
                fire_gathers(r ^ 1)
                compute(c, r)
            return 0

        lax.fori_loop(0, NCHUNK // NBUF, step, 0)
        drain_gathers(0)
        wait_idx(1)

        pltpu.sync_copy(ln_all, out_near.at[pl.ds(base * P, BPT * P)])
        pltpu.sync_copy(lg_all, out_neg.at[pl.ds(base * N, BPT * N)])

    return k


def _tc_loss(ln, lg):
    B, P = ln.shape
    N = lg.shape[1]
    BLK = 2048

    def body(ln_ref, lg_ref, out_ref):
        def lsig(x):
            return jnp.minimum(x, 0.0) - jnp.log1p(jnp.exp(-jnp.abs(x)))
        out_ref[...] = -(lsig(ln_ref[...]).sum(axis=1)
                         + lsig(-lg_ref[...]).sum(axis=1))

    return pl.pallas_call(
        body,
        grid=(B // BLK,),
        in_specs=[
            pl.BlockSpec((BLK, P), lambda i: (i, 0)),
            pl.BlockSpec((BLK, N), lambda i: (i, 0)),
        ],
        out_specs=pl.BlockSpec((BLK,), lambda i: (i,)),
        out_shape=jax.ShapeDtypeStruct((B,), jnp.float32),
    )(ln, lg)


def kernel(input_wordids, near_wordids, neg_wordids, input_weight):
    B, P = near_wordids.shape
    N = neg_wordids.shape[1]
    V, D = input_weight.shape
    W = 8
    GCH = 80
    UNROLL = 5

    ids = input_wordids.astype(jnp.int32)
    near = near_wordids.reshape(B * P).astype(jnp.int32)
    neg = neg_wordids.reshape(B * N).astype(jnp.int32)

    ln, lg = _sc_logits(B, P, N, V, D, W, GCH, UNROLL)(ids, near, neg,
                                                       input_weight)
    return _tc_loss(ln.reshape(B, P), lg.reshape(B, N))


# speedup vs baseline: 10.4792x; 1.1468x over previous
import functools

import jax
import jax.numpy as jnp
from jax import lax
from jax.experimental import pallas as pl
from jax.experimental.pallas import tpu as pltpu
from jax.experimental.pallas import tpu_sc as plsc

NC = 2
NS = 16
NW = NC * NS
LANES = 16


def _sc_logits(B, P, N, V, D, W, GCH, UNROLL):
    BPT = B // NW
    NCHUNK = BPT // W
    NBUF = 2
    WP, WN = W * P, W * N

    mesh = plsc.VectorSubcoreMesh(core_axis_name="c", subcore_axis_name="s")

    scratch = []
    for _ in range(NBUF):
        scratch += [
            pltpu.VMEM((W,), jnp.int32),
            pltpu.VMEM((WP,), jnp.int32),
            pltpu.VMEM((WN,), jnp.int32),
            pltpu.VMEM((W, D), jnp.float32),
            pltpu.VMEM((WP, D), jnp.float32),
            pltpu.VMEM((WN, D), jnp.float32),
            pltpu.SemaphoreType.DMA,
            pltpu.SemaphoreType.DMA,
        ]
    scratch += [
        pltpu.VMEM((BPT * P,), jnp.float32),
        pltpu.VMEM((BPT * N,), jnp.float32),
    ]

    @functools.partial(
        pl.kernel,
        out_type=(
            jax.ShapeDtypeStruct((B * P,), jnp.float32),
            jax.ShapeDtypeStruct((B * N,), jnp.float32),
        ),
        mesh=mesh,
        scratch_types=scratch,
        compiler_params=pltpu.CompilerParams(needs_layout_passes=False,
                                             use_tc_tiling_on_sc=False),
    )
    def k(inp_ids, near_ids, neg_ids, table, out_near, out_neg, *s):
        bufs = [s[i * 8:(i + 1) * 8] for i in range(NBUF)]
        ln_all, lg_all = s[NBUF * 8], s[NBUF * 8 + 1]
        wid = lax.axis_index("s") * NC + lax.axis_index("c")
        base = wid * BPT
        LAST = NCHUNK - 1

        def fire_idx(c, r):
            idx_i, idx_p, idx_n = bufs[r][0:3]
            isem = bufs[r][7]
            b0 = base + c * W
            pltpu.async_copy(inp_ids.at[pl.ds(b0, W)], idx_i, isem)
            pltpu.async_copy(near_ids.at[pl.ds(b0 * P, WP)], idx_p, isem)
            pltpu.async_copy(neg_ids.at[pl.ds(b0 * N, WN)], idx_n, isem)

        def wait_idx(r):
            idx_i, idx_p, idx_n = bufs[r][0:3]
            isem = bufs[r][7]
            pltpu.make_async_copy(inp_ids.at[pl.ds(0, W)], idx_i, isem).wait()
            pltpu.make_async_copy(near_ids.at[pl.ds(0, WP)], idx_p, isem).wait()
            pltpu.make_async_copy(neg_ids.at[pl.ds(0, WN)], idx_n, isem).wait()

        def fire_gathers(r):
            idx_i, idx_p, idx_n, rows_i, rows_p, rows_n, gsem, _ = bufs[r]
            pltpu.async_copy(table.at[idx_i], rows_i, gsem)
            for o in range(0, WP, GCH):
                pltpu.async_copy(table.at[idx_p.at[pl.ds(o, GCH)]],
                                 rows_p.at[pl.ds(o, GCH)], gsem)
            for o in range(0, WN, GCH):
                pltpu.async_copy(table.at[idx_n.at[pl.ds(o, GCH)]],
                                 rows_n.at[pl.ds(o, GCH)], gsem)

        def drain_gathers(r):
            idx_i, idx_p, idx_n, rows_i, rows_p, rows_n, gsem, _ = bufs[r]
            pltpu.make_async_copy(table.at[idx_i], rows_i, gsem).wait()
            for o in range(0, WP, GCH):
                pltpu.make_async_copy(table.at[idx_p.at[pl.ds(o, GCH)]],
                                      rows_p.at[pl.ds(o, GCH)], gsem).wait()
            for o in range(0, WN, GCH):
                pltpu.make_async_copy(table.at[idx_n.at[pl.ds(o, GCH)]],
                                      rows_n.at[pl.ds(o, GCH)], gsem).wait()

        lane = lax.iota(jnp.int32, LANES)
        last_lane = lane == (LANES - 1)

        def compute(c, r):
            rows_i, rows_p, rows_n = bufs[r][3:6]
            for b in range(W):
                ivecs = [rows_i[b, pl.ds(j * LANES, LANES)]
                         for j in range(D // LANES)]

                def dots(K, rows, out, obase):
                    @plsc.parallel_loop(0, K, 1, unroll=UNROLL)
                    def _(kk):
                        row = b * K + kk
                        acc = rows[row, pl.ds(0, LANES)] * ivecs[0]
                        for j in range(1, D // LANES):
                            acc = acc + (rows[row, pl.ds(j * LANES, LANES)]
                                         * ivecs[j])
                        s_ = plsc.cumsum(acc)
                        oidx = jnp.full((LANES,), obase + kk, jnp.int32)
                        plsc.store_scatter(out, [oidx], s_, mask=last_lane)

                dots(P, rows_p, ln_all, (c * W + b) * P)
                dots(N, rows_n, lg_all, (c * W + b) * N)

        fire_idx(0, 0)
        wait_idx(0)
        fire_gathers(0)
        fire_idx(1, 1)

        def step(i, _):
            for r in range(NBUF):
                c = i * NBUF + r
                drain_gathers(r)
                fire_idx(jnp.minimum(c + 2, LAST), r)
                wait_idx(r ^ 1)
                fire_gathers(r ^ 1)
                compute(c, r)
            return 0

        lax.fori_loop(0, NCHUNK // NBUF, step, 0)
        drain_gathers(0)
        wait_idx(1)

        pltpu.sync_copy(ln_all, out_near.at[pl.ds(base * P, BPT * P)])
        pltpu.sync_copy(lg_all, out_neg.at[pl.ds(base * N, BPT * N)])

    return k


def _tc_linear_table(table, VB=2048):
    V, D = table.shape

    def body(in_ref, out_ref):
        xt = in_ref[...].T
        out_ref[...] = jnp.concatenate([xt, jnp.zeros_like(xt)], axis=1)

    out = pl.pallas_call(
        body,
        grid=(-(-V // VB),),
        in_specs=[pl.BlockSpec((D, VB), lambda i: (0, i))],
        out_specs=pl.BlockSpec((VB, 2 * D), lambda i: (i, 0)),
        out_shape=jax.ShapeDtypeStruct((V, 2 * D), jnp.float32),
    )(table.T)
    return out.reshape(2 * V, D)


def _tc_loss(ln, lg):
    B, P = ln.shape
    N = lg.shape[1]
    BLK = 2048

    def body(ln_ref, lg_ref, out_ref):
        def lsig(x):
            return jnp.minimum(x, 0.0) - jnp.log1p(jnp.exp(-jnp.abs(x)))
        out_ref[...] = -(lsig(ln_ref[...]).sum(axis=1)
                         + lsig(-lg_ref[...]).sum(axis=1))

    return pl.pallas_call(
        body,
        grid=(B // BLK,),
        in_specs=[
            pl.BlockSpec((BLK, P), lambda i: (i, 0)),
            pl.BlockSpec((BLK, N), lambda i: (i, 0)),
        ],
        out_specs=pl.BlockSpec((BLK,), lambda i: (i,)),
        out_shape=jax.ShapeDtypeStruct((B,), jnp.float32),
    )(ln, lg)


def kernel(input_wordids, near_wordids, neg_wordids, input_weight):
    B, P = near_wordids.shape
    N = neg_wordids.shape[1]
    V, D = input_weight.shape
    W = 8
    GCH = 80
    UNROLL = 5

    ids = input_wordids.astype(jnp.int32) * 2
    near = near_wordids.reshape(B * P).astype(jnp.int32) * 2
    neg = neg_wordids.reshape(B * N).astype(jnp.int32) * 2
    table_lin = _tc_linear_table(input_weight)

    ln, lg = _sc_logits(B, P, N, 2 * V, D, W, GCH, UNROLL)(ids, near, neg,
                                                           table_lin)
    return _tc_loss(ln.reshape(B, P), lg.reshape(B, N))


# speedup vs baseline: 14.6048x; 1.3937x over previous
import functools

import jax
import jax.numpy as jnp
from jax import lax
from jax.experimental import pallas as pl
from jax.experimental.pallas import tpu as pltpu
from jax.experimental.pallas import tpu_sc as plsc

NC = 2
NS = 16
NW = NC * NS
LANES = 16


def _sc_logits(B, P, N, V, D, W, GCH, UNROLL):
    BPT = B // NW
    NCHUNK = BPT // W
    NBUF = 2
    WP, WN = W * P, W * N

    mesh = plsc.VectorSubcoreMesh(core_axis_name="c", subcore_axis_name="s")

    scratch = []
    for _ in range(NBUF):
        scratch += [
            pltpu.VMEM((W,), jnp.int32),
            pltpu.VMEM((WP,), jnp.int32),
            pltpu.VMEM((WN,), jnp.int32),
            pltpu.VMEM((W, D), jnp.float32),
            pltpu.VMEM((WP, D), jnp.float32),
            pltpu.VMEM((WN, D), jnp.float32),
            pltpu.SemaphoreType.DMA,
            pltpu.SemaphoreType.DMA,
        ]
    scratch += [
        pltpu.VMEM((BPT * P,), jnp.float32),
        pltpu.VMEM((BPT * N,), jnp.float32),
    ]

    @functools.partial(
        pl.kernel,
        out_type=(
            jax.ShapeDtypeStruct((B * P,), jnp.float32),
            jax.ShapeDtypeStruct((B * N,), jnp.float32),
        ),
        mesh=mesh,
        scratch_types=scratch,
        compiler_params=pltpu.CompilerParams(needs_layout_passes=False,
                                             use_tc_tiling_on_sc=False),
    )
    def k(inp_ids, near_ids, neg_ids, table, out_near, out_neg, *s):
        bufs = [s[i * 8:(i + 1) * 8] for i in range(NBUF)]
        ln_all, lg_all = s[NBUF * 8], s[NBUF * 8 + 1]
        wid = lax.axis_index("s") * NC + lax.axis_index("c")
        base = wid * BPT
        LAST = NCHUNK - 1

        def fire_idx(c, r):
            idx_i, idx_p, idx_n = bufs[r][0:3]
            isem = bufs[r][7]
            b0 = base + c * W
            pltpu.async_copy(inp_ids.at[pl.ds(b0, W)], idx_i, isem)
            pltpu.async_copy(near_ids.at[pl.ds(b0 * P, WP)], idx_p, isem)
            pltpu.async_copy(neg_ids.at[pl.ds(b0 * N, WN)], idx_n, isem)

        def wait_idx(r):
            idx_i, idx_p, idx_n = bufs[r][0:3]
            isem = bufs[r][7]
            pltpu.make_async_copy(inp_ids.at[pl.ds(0, W)], idx_i, isem).wait()
            pltpu.make_async_copy(near_ids.at[pl.ds(0, WP)], idx_p, isem).wait()
            pltpu.make_async_copy(neg_ids.at[pl.ds(0, WN)], idx_n, isem).wait()

        def fire_gathers(r):
            idx_i, idx_p, idx_n, rows_i, rows_p, rows_n, gsem, _ = bufs[r]
            pltpu.async_copy(table.at[idx_i], rows_i, gsem)
            for o in range(0, WP, GCH):
                pltpu.async_copy(table.at[idx_p.at[pl.ds(o, GCH)]],
                                 rows_p.at[pl.ds(o, GCH)], gsem)
            for o in range(0, WN, GCH):
                pltpu.async_copy(table.at[idx_n.at[pl.ds(o, GCH)]],
                                 rows_n.at[pl.ds(o, GCH)], gsem)

        def drain_gathers(r):
            idx_i, idx_p, idx_n, rows_i, rows_p, rows_n, gsem, _ = bufs[r]
            pltpu.make_async_copy(table.at[idx_i], rows_i, gsem).wait()
            for o in range(0, WP, GCH):
                pltpu.make_async_copy(table.at[idx_p.at[pl.ds(o, GCH)]],
                                      rows_p.at[pl.ds(o, GCH)], gsem).wait()
            for o in range(0, WN, GCH):
                pltpu.make_async_copy(table.at[idx_n.at[pl.ds(o, GCH)]],
                                      rows_n.at[pl.ds(o, GCH)], gsem).wait()

        lane = lax.iota(jnp.int32, LANES)
        last_lane = lane == (LANES - 1)

        def compute(c, r):
            rows_i, rows_p, rows_n = bufs[r][3:6]
            for b in range(W):
                ivecs = [rows_i[b, pl.ds(j * LANES, LANES)]
                         for j in range(D // LANES)]

                def dots(K, rows, out, obase):
                    @plsc.parallel_loop(0, K, 1, unroll=UNROLL)
                    def _(kk):
                        row = b * K + kk
                        acc = rows[row, pl.ds(0, LANES)] * ivecs[0]
                        for j in range(1, D // LANES):
                            acc = acc + (rows[row, pl.ds(j * LANES, LANES)]
                                         * ivecs[j])
                        s_ = plsc.cumsum(acc)
                        oidx = jnp.full((LANES,), obase + kk, jnp.int32)
                        plsc.store_scatter(out, [oidx], s_, mask=last_lane)

                dots(P, rows_p, ln_all, (c * W + b) * P)
                dots(N, rows_n, lg_all, (c * W + b) * N)

        fire_idx(0, 0)
        wait_idx(0)
        fire_gathers(0)
        fire_idx(1, 1)

        def step(i, _):
            for r in range(NBUF):
                c = i * NBUF + r
                drain_gathers(r)
                fire_idx(jnp.minimum(c + 2, LAST), r)
                wait_idx(r ^ 1)
                fire_gathers(r ^ 1)
                compute(c, r)
            return 0

        lax.fori_loop(0, NCHUNK // NBUF, step, 0)
        drain_gathers(0)
        wait_idx(1)

        pltpu.sync_copy(ln_all, out_near.at[pl.ds(base * P, BPT * P)])
        pltpu.sync_copy(lg_all, out_neg.at[pl.ds(base * N, BPT * N)])

    return k


_VB = 8192
_VBH = _VB // 2


def _tc_linear_table(table):
    V, D = table.shape
    grid = -(-V // _VB)

    def body(in_ref, out_ref):
        xt = in_ref[...].T
        out_ref[...] = jnp.concatenate([xt[:_VBH], xt[_VBH:]], axis=1)

    out = pl.pallas_call(
        body,
        grid=(grid,),
        in_specs=[pl.BlockSpec((D, _VB), lambda i: (0, i))],
        out_specs=pl.BlockSpec((_VBH, 2 * D), lambda i: (i, 0)),
        out_shape=jax.ShapeDtypeStruct((grid * _VBH, 2 * D), jnp.float32),
    )(table.T)
    return out.reshape(grid * _VB, D)


def _map_ids(v):
    return (v & -_VB) + ((v & (_VBH - 1)) << 1) + ((v >> (_VBH.bit_length() - 1)) & 1)


def _tc_loss(ln, lg):
    B, P = ln.shape
    N = lg.shape[1]
    BLK = 2048

    def body(ln_ref, lg_ref, out_ref):
        def lsig(x):
            return jnp.minimum(x, 0.0) - jnp.log1p(jnp.exp(-jnp.abs(x)))
        out_ref[...] = -(lsig(ln_ref[...]).sum(axis=1)
                         + lsig(-lg_ref[...]).sum(axis=1))

    return pl.pallas_call(
        body,
        grid=(B // BLK,),
        in_specs=[
            pl.BlockSpec((BLK, P), lambda i: (i, 0)),
            pl.BlockSpec((BLK, N), lambda i: (i, 0)),
        ],
        out_specs=pl.BlockSpec((BLK,), lambda i: (i,)),
        out_shape=jax.ShapeDtypeStruct((B,), jnp.float32),
    )(ln, lg)


def kernel(input_wordids, near_wordids, neg_wordids, input_weight):
    B, P = near_wordids.shape
    N = neg_wordids.shape[1]
    V, D = input_weight.shape
    W = 8
    GCH = 80
    UNROLL = 5

    ids = _map_ids(input_wordids.astype(jnp.int32))
    near = _map_ids(near_wordids.reshape(B * P).astype(jnp.int32))
    neg = _map_ids(neg_wordids.reshape(B * N).astype(jnp.int32))
    table_lin = _tc_linear_table(input_weight)

    ln, lg = _sc_logits(B, P, N, table_lin.shape[0], D, W, GCH, UNROLL)(
        ids, near, neg, table_lin)
    return _tc_loss(ln.reshape(B, P), lg.reshape(B, N))
